# baseline (speedup 1.0000x reference)
import functools

import jax
import jax.numpy as jnp
from jax import lax
from jax.experimental import pallas as pl
from jax.experimental.pallas import tpu as pltpu
from jax.experimental.pallas import tpu_sc as plsc

_C = 150
_H = 512
_W = 512
_HW = _H * _W

_NC = 2
_NS = 16
_L = 16
_NW = _NC * _NS
_RPW = _H // _NW
_PPW = _HW // _NW
_CCH = 7
_NCHUNK = -(-_C // _CCH)
_BROWS = _CCH * _RPW

_mesh = plsc.VectorSubcoreMesh(
    core_axis_name="c", subcore_axis_name="s",
    num_cores=_NC, num_subcores=_NS)


@functools.partial(
    pl.kernel,
    out_type=jax.ShapeDtypeStruct((_C, _H, _W), jnp.float32),
    mesh=_mesh,
    compiler_params=pltpu.CompilerParams(
        needs_layout_passes=False,
        disable_bounds_checks=True,
        disable_semaphore_checks=True,
    ),
    scratch_types=[
        pltpu.VMEM((_RPW, _W), jnp.int32),
        pltpu.VMEM((_BROWS, _W), jnp.float32),
        pltpu.VMEM((_BROWS, _W), jnp.float32),
        pltpu.SemaphoreType.DMA,
        pltpu.SemaphoreType.DMA,
    ],
)
def _sc_onehot(lab_hbm, out_hbm, r_v, buf_a, buf_b, sem_a, sem_b):
    wid = lax.axis_index("s") * _NC + lax.axis_index("c")
    row0 = wid * _RPW

    lab_hnd = pltpu.async_copy(lab_hbm.at[pl.ds(row0, _RPW), :], r_v, sem_a)

    lane = lax.iota(jnp.int32, _L)
    ones = jnp.full((_L,), 1.0, jnp.float32)
    zeros16 = jnp.zeros((_L,), jnp.float32)
    bound = jnp.uint32(_BROWS)

    def full_clear(buf):
        @plsc.parallel_loop(0, _BROWS * _W, step=_L, unroll=8)
        def _clr(q0):
            buf[jax.lax.shift_right_logical(q0, 9),
                pl.ds(q0 & (_W - 1), _L)] = zeros16

    full_clear(buf_a)
    full_clear(buf_b)
    lab_hnd.wait()

    @plsc.parallel_loop(0, _PPW, step=_L, unroll=8)
    def _init_r(q0):
        h = jax.lax.shift_right_logical(q0, 9)
        sl = pl.ds(q0 & (_W - 1), _L)
        r_v[h, sl] = r_v[h, sl] * _RPW + h

    def scan_chunk(buf, roff_new, roff_old):
        @plsc.parallel_loop(0, _PPW, step=_L, unroll=8)
        def _scan(q0):
            rv = r_v[jax.lax.shift_right_logical(q0, 9),
                     pl.ds(q0 & (_W - 1), _L)]
            colv = (q0 & (_W - 1)) + lane
            row_new = rv - roff_new
            msk_new = plsc.bitcast(row_new, jnp.uint32) < bound
            plsc.store_scatter(buf, [row_new, colv], ones, mask=msk_new)
            if roff_old is not None:
                row_old = rv - roff_old
                msk_old = plsc.bitcast(row_old, jnp.uint32) < bound
                plsc.store_scatter(buf, [row_old, colv], zeros16, mask=msk_old)

    bufs = (buf_a, buf_b)
    sems = (sem_a, sem_b)
    pending = [None, None]
    for k in range(_NCHUNK):
        b = k % 2
        buf = bufs[b]
        if pending[b] is not None:
            for hnd in pending[b]:
                hnd.wait()
            scan_chunk(buf, k * _BROWS, (k - 2) * _BROWS)
        else:
            scan_chunk(buf, k * _BROWS, None)

        hnds = []
        for j in range(_CCH):
            c = k * _CCH + j
            if c >= _C:
                break
            hnds.append(pltpu.async_copy(
                buf.at[pl.ds(j * _RPW, _RPW), :],
                out_hbm.at[c, pl.ds(row0, _RPW), :],
                sems[b]))
        pending[b] = hnds

    for b in range(2):
        for hnd in pending[b]:
            hnd.wait()


def kernel(label):
    return _sc_onehot(label)


# speedup vs baseline: 1.0113x; 1.0113x over previous
import functools

import jax
import jax.numpy as jnp
from jax import lax
from jax.experimental import pallas as pl
from jax.experimental.pallas import tpu as pltpu
from jax.experimental.pallas import tpu_sc as plsc

_C = 150
_H = 512
_W = 512
_HW = _H * _W

_NC = 2
_NS = 16
_L = 16
_NW = _NC * _NS
_RPW = _H // _NW
_PPW = _HW // _NW
_CCH = 7
_NCHUNK = -(-_C // _CCH)
_BROWS = _CCH * _RPW

_mesh = plsc.VectorSubcoreMesh(
    core_axis_name="c", subcore_axis_name="s",
    num_cores=_NC, num_subcores=_NS)


@functools.partial(
    pl.kernel,
    out_type=jax.ShapeDtypeStruct((_C, _H, _W), jnp.float32),
    mesh=_mesh,
    compiler_params=pltpu.CompilerParams(needs_layout_passes=False),
    scratch_types=[
        pltpu.VMEM((_RPW, _W), jnp.int32),
        pltpu.VMEM((_CCH, _RPW, _W), jnp.float32),
        pltpu.VMEM((_CCH, _RPW, _W), jnp.float32),
        pltpu.SemaphoreType.DMA,
        pltpu.SemaphoreType.DMA,
    ],
)
def _sc_onehot(lab_hbm, out_hbm, r_v, buf_a, buf_b, sem_a, sem_b):
    wid = lax.axis_index("s") * _NC + lax.axis_index("c")
    row0 = wid * _RPW

    lab_hnd = pltpu.async_copy(lab_hbm.at[pl.ds(row0, _RPW), :], r_v, sem_a)

    lane = lax.iota(jnp.int32, _L)
    ones = jnp.full((_L,), 1.0, jnp.float32)
    zeros16 = jnp.zeros((_L,), jnp.float32)
    bound = jnp.uint32(_BROWS)

    def full_clear(buf):
        @plsc.parallel_loop(0, _BROWS * _W, step=_L, unroll=8)
        def _clr(q0):
            r = jax.lax.shift_right_logical(q0, 9)
            buf[jax.lax.shift_right_logical(r, 4), r & (_RPW - 1),
                pl.ds(q0 & (_W - 1), _L)] = zeros16

    full_clear(buf_a)
    full_clear(buf_b)
    lab_hnd.wait()

    @plsc.parallel_loop(0, _PPW, step=_L, unroll=8)
    def _init_r(q0):
        h = jax.lax.shift_right_logical(q0, 9)
        sl = pl.ds(q0 & (_W - 1), _L)
        r_v[h, sl] = r_v[h, sl] * _RPW + h

    def scan_chunk(buf, roff_new, roff_old):
        @plsc.parallel_loop(0, _PPW, step=_L, unroll=8)
        def _scan(q0):
            rv = r_v[jax.lax.shift_right_logical(q0, 9),
                     pl.ds(q0 & (_W - 1), _L)]
            colv = (q0 & (_W - 1)) + lane
            row_new = rv - roff_new
            msk_new = plsc.bitcast(row_new, jnp.uint32) < bound
            plsc.store_scatter(
                buf,
                [jax.lax.shift_right_logical(row_new, 4),
                 row_new & (_RPW - 1), colv],
                ones, mask=msk_new)
            if roff_old is not None:
                row_old = rv - roff_old
                msk_old = plsc.bitcast(row_old, jnp.uint32) < bound
                plsc.store_scatter(
                    buf,
                    [jax.lax.shift_right_logical(row_old, 4),
                     row_old & (_RPW - 1), colv],
                    zeros16, mask=msk_old)

    bufs = (buf_a, buf_b)
    sems = (sem_a, sem_b)
    pending = [None, None]
    for k in range(_NCHUNK):
        b = k % 2
        buf = bufs[b]
        if pending[b] is not None:
            for hnd in pending[b]:
                hnd.wait()
            scan_chunk(buf, k * _BROWS, (k - 2) * _BROWS)
        else:
            scan_chunk(buf, k * _BROWS, None)

        c0 = k * _CCH
        if c0 + _CCH <= _C:
            hnds = [pltpu.async_copy(
                buf,
                out_hbm.at[pl.ds(c0, _CCH), pl.ds(row0, _RPW), :],
                sems[b])]
        else:
            hnds = [pltpu.async_copy(
                buf.at[j],
                out_hbm.at[c0 + j, pl.ds(row0, _RPW), :],
                sems[b]) for j in range(_C - c0)]
        pending[b] = hnds

    for b in range(2):
        for hnd in pending[b]:
            hnd.wait()


def kernel(label):
    return _sc_onehot(label)


# speedup vs baseline: 1.0118x; 1.0005x over previous
import functools

import jax
import jax.numpy as jnp
from jax import lax
from jax.experimental import pallas as pl
from jax.experimental.pallas import tpu as pltpu
from jax.experimental.pallas import tpu_sc as plsc

_C = 150
_H = 512
_W = 512
_HW = _H * _W

_NC = 2
_NS = 16
_L = 16
_NW = _NC * _NS
_RPW = _H // _NW
_PPW = _HW // _NW
_CCH = 7
_NCHUNK = -(-_C // _CCH)
_BROWS = _CCH * _RPW

_mesh = plsc.VectorSubcoreMesh(
    core_axis_name="c", subcore_axis_name="s",
    num_cores=_NC, num_subcores=_NS)


@functools.partial(
    pl.kernel,
    out_type=jax.ShapeDtypeStruct((_C, _H, _W), jnp.float32),
    mesh=_mesh,
    compiler_params=pltpu.CompilerParams(
        needs_layout_passes=False, skip_device_barrier=True),
    scratch_types=[
        pltpu.VMEM((_RPW, _W), jnp.int32),
        pltpu.VMEM((_CCH, _RPW, _W), jnp.float32),
        pltpu.VMEM((_CCH, _RPW, _W), jnp.float32),
        pltpu.SemaphoreType.DMA,
        pltpu.SemaphoreType.DMA,
    ],
)
def _sc_onehot(lab_hbm, out_hbm, r_v, buf_a, buf_b, sem_a, sem_b):
    wid = lax.axis_index("s") * _NC + lax.axis_index("c")
    row0 = wid * _RPW

    lab_hnd = pltpu.async_copy(lab_hbm.at[pl.ds(row0, _RPW), :], r_v, sem_a)

    lane = lax.iota(jnp.int32, _L)
    ones = jnp.full((_L,), 1.0, jnp.float32)
    zeros16 = jnp.zeros((_L,), jnp.float32)
    bound = jnp.uint32(_BROWS)

    def full_clear(buf):
        @plsc.parallel_loop(0, _BROWS * _W, step=_L, unroll=8)
        def _clr(q0):
            r = jax.lax.shift_right_logical(q0, 9)
            buf[jax.lax.shift_right_logical(r, 4), r & (_RPW - 1),
                pl.ds(q0 & (_W - 1), _L)] = zeros16

    full_clear(buf_a)
    full_clear(buf_b)
    lab_hnd.wait()

    @plsc.parallel_loop(0, _PPW, step=_L, unroll=8)
    def _init_r(q0):
        h = jax.lax.shift_right_logical(q0, 9)
        sl = pl.ds(q0 & (_W - 1), _L)
        r_v[h, sl] = r_v[h, sl] * _RPW + h

    def scan_chunk(buf, roff_new, roff_old):
        @plsc.parallel_loop(0, _PPW, step=_L, unroll=8)
        def _scan(q0):
            rv = r_v[jax.lax.shift_right_logical(q0, 9),
                     pl.ds(q0 & (_W - 1), _L)]
            colv = (q0 & (_W - 1)) + lane
            row_new = rv - roff_new
            msk_new = plsc.bitcast(row_new, jnp.uint32) < bound
            plsc.store_scatter(
                buf,
                [jax.lax.shift_right_logical(row_new, 4),
                 row_new & (_RPW - 1), colv],
                ones, mask=msk_new)
            if roff_old is not None:
                row_old = rv - roff_old
                msk_old = plsc.bitcast(row_old, jnp.uint32) < bound
                plsc.store_scatter(
                    buf,
                    [jax.lax.shift_right_logical(row_old, 4),
                     row_old & (_RPW - 1), colv],
                    zeros16, mask=msk_old)

    bufs = (buf_a, buf_b)
    sems = (sem_a, sem_b)
    pending = [None, None]
    for k in range(_NCHUNK):
        b = k % 2
        buf = bufs[b]
        if pending[b] is not None:
            for hnd in pending[b]:
                hnd.wait()
            scan_chunk(buf, k * _BROWS, (k - 2) * _BROWS)
        else:
            scan_chunk(buf, k * _BROWS, None)

        c0 = k * _CCH
        if c0 + _CCH <= _C:
            hnds = [pltpu.async_copy(
                buf,
                out_hbm.at[pl.ds(c0, _CCH), pl.ds(row0, _RPW), :],
                sems[b])]
        else:
            hnds = [pltpu.async_copy(
                buf.at[j],
                out_hbm.at[c0 + j, pl.ds(row0, _RPW), :],
                sems[b]) for j in range(_C - c0)]
        pending[b] = hnds

    for b in range(2):
        for hnd in pending[b]:
            hnd.wait()


def kernel(label):
    return _sc_onehot(label)
